# Initial kernel scaffold; baseline (speedup 1.0000x reference)
import jax
import jax.numpy as jnp
from jax.experimental import pallas as pl


def kernel(x, edge_index, edge_attr, batch, atom_tables, bond_tables, conv_eps, conv_W1, conv_b1, conv_bn_g, conv_bn_b, conv_W2, conv_b2, node_bn_g, node_bn_b, vn_embed, vn_W1, vn_b1, vn_bn1_g, vn_bn1_b, vn_W2, vn_b2, vn_bn2_g, vn_bn2_b, pred_W, pred_b):
    raise NotImplementedError("write your pallas kernel here")


# speedup vs baseline: 3.7366x; 3.7366x over previous
import functools

import jax
import jax.numpy as jnp
from jax import lax
from jax.experimental import pallas as pl
from jax.experimental.pallas import tpu as pltpu
from jax.experimental.pallas import tpu_sc as plsc

N = 10000
E = 160000
D = 256
H = 512
L = 5
G = 64
T = 128
BN_INV = 1.0 / (1.0 + 1e-5) ** 0.5

BN_NODES = 1000
N_BLOCKS = N // BN_NODES

NC = 2
NS = 16
EDGES_PER_TILE = E // NS
EB = 80
NB_EDGE = EDGES_PER_TILE // EB
ROWS_PER_TILE = 624
ROWS_TAIL = N - NS * ROWS_PER_TILE


def _onehot(idx, num):
  iota = lax.broadcasted_iota(jnp.int32, (1, num), 1)
  return (idx[:, None] == iota).astype(jnp.float32)


def _sc_edge_body(h2, ct2, src, dst, combo, out, idx_h, idx_e, dst_v,
                  rows_h, rows_e, acc, sem_h, sem_e):
  c = lax.axis_index("c")
  s = lax.axis_index("s")

  z = jnp.zeros((16,), jnp.float32)

  def zero_row(r, _):
    for j in range(8):
      rows_e[r, pl.ds(j * 16, 16)] = z
    return 0

  lax.fori_loop(0, EB, zero_row, 0)

  if True:
    row0 = s * ROWS_PER_TILE
    for j in range(ROWS_PER_TILE // EB):
      pltpu.sync_copy(rows_e, acc.at[pl.ds(row0 + j * EB, EB)])
    rem = ROWS_PER_TILE % EB
    pltpu.sync_copy(rows_e.at[pl.ds(0, rem)],
                    acc.at[pl.ds(row0 + (ROWS_PER_TILE // EB) * EB, rem)])

    @pl.when(s == NS - 1)
    def _():
      pltpu.sync_copy(rows_e.at[pl.ds(0, ROWS_TAIL)],
                      acc.at[pl.ds(NS * ROWS_PER_TILE, ROWS_TAIL)])

    plsc.subcore_barrier()

    ebase = s * EDGES_PER_TILE

    def block(i, _):
      base = ebase + i * EB
      pltpu.sync_copy(src.at[pl.ds(base, EB)], idx_h)
      pltpu.sync_copy(combo.at[pl.ds(base, EB)], idx_e)
      pltpu.sync_copy(dst.at[pl.ds(base, EB)], dst_v)
      for k in range(EB // 16):
        sl = pl.ds(k * 16, 16)
        idx_h[sl] = idx_h[sl] * 2 + c
        idx_e[sl] = idx_e[sl] * 2 + c
      cp_h = pltpu.async_copy(h2.at[idx_h], rows_h, sem_h)
      cp_e = pltpu.async_copy(ct2.at[idx_e], rows_e, sem_e)
      cp_h.wait()
      cp_e.wait()

      def relu_row(r, _):
        for j in range(8):
          sl = pl.ds(j * 16, 16)
          rows_h[r, sl] = jnp.maximum(rows_h[r, sl] + rows_e[r, sl], 0.0)
        return 0

      lax.fori_loop(0, EB, relu_row, 0)
      pltpu.sync_copy(rows_h, acc.at[dst_v], add=True)
      return 0

    lax.fori_loop(0, NB_EDGE, block, 0)
    plsc.subcore_barrier()

    pltpu.sync_copy(acc.at[pl.ds(row0, ROWS_PER_TILE)],
                    out.at[c, pl.ds(row0, ROWS_PER_TILE)])

    @pl.when(s == NS - 1)
    def _():
      pltpu.sync_copy(acc.at[pl.ds(NS * ROWS_PER_TILE, ROWS_TAIL)],
                      out.at[c, pl.ds(NS * ROWS_PER_TILE, ROWS_TAIL)])


@functools.partial(jax.jit, static_argnums=())
def _sc_edge(h2, ct2, src, dst, combo):
  fn = pl.kernel(
      _sc_edge_body,
      out_type=jax.ShapeDtypeStruct((NC, N, D // NC), jnp.float32),
      mesh=plsc.VectorSubcoreMesh(core_axis_name="c", subcore_axis_name="s"),
      scratch_types=[
          pltpu.VMEM((EB,), jnp.int32),
          pltpu.VMEM((EB,), jnp.int32),
          pltpu.VMEM((EB,), jnp.int32),
          pltpu.VMEM((EB, D // NC), jnp.float32),
          pltpu.VMEM((EB, D // NC), jnp.float32),
          pltpu.VMEM_SHARED((N, D // NC), jnp.float32),
          pltpu.SemaphoreType.DMA,
          pltpu.SemaphoreType.DMA,
      ],
  )
  return fn(h2, ct2, src, dst, combo)


def _combo_body(ea0, ea1, ea2, out):
  out[...] = ea0[...] * 64 + ea1[...] * 8 + ea2[...]


def _combo(ea0, ea1, ea2):
  return pl.pallas_call(
      _combo_body,
      out_shape=jax.ShapeDtypeStruct((E // 128, 128), jnp.int32),
  )(ea0, ea1, ea2)


def _ct_body(bt, out):
  i512 = lax.broadcasted_iota(jnp.int32, (512,), 0)
  oh0 = _onehot(i512 // 64, 8)
  oh1 = _onehot((i512 // 8) % 8, 8)
  oh2 = _onehot(i512 % 8, 8)
  r = jnp.dot(oh0, bt[0, 0], preferred_element_type=jnp.float32)
  r += jnp.dot(oh1, bt[0, 1], preferred_element_type=jnp.float32)
  r += jnp.dot(oh2, bt[0, 2], preferred_element_type=jnp.float32)
  out[0] = r


def _ct(bond_tables):
  return pl.pallas_call(
      _ct_body,
      grid=(L,),
      in_specs=[pl.BlockSpec((1, 3, 8, D), lambda l: (l, 0, 0, 0))],
      out_specs=pl.BlockSpec((1, 512, D), lambda l: (l, 0, 0)),
      out_shape=jax.ShapeDtypeStruct((L, 512, D), jnp.float32),
  )(bond_tables)


def _atom_body(xb, batchb, tabs, vne, h_out, gsum, counts):
  i = pl.program_id(0)
  h = jnp.dot(_onehot(xb[:, 0], 64), tabs[0],
              preferred_element_type=jnp.float32)
  for j in range(1, 9):
    h += jnp.dot(_onehot(xb[:, j], 64), tabs[j],
                 preferred_element_type=jnp.float32)
  h += vne[...]
  h_out[...] = h
  oh = _onehot(batchb[0, 0, :], G)

  @pl.when(i == 0)
  def _():
    gsum[...] = jnp.zeros_like(gsum)
    counts[...] = jnp.zeros_like(counts)

  gsum[...] += lax.dot_general(oh, h, (((0,), (0,)), ((), ())),
                               preferred_element_type=jnp.float32)
  counts[...] += lax.dot_general(
      oh, jnp.ones((BN_NODES, 128), jnp.float32),
      (((0,), (0,)), ((), ())), preferred_element_type=jnp.float32)


def _atom(x, batch3, atom_tables, vn_embed):
  return pl.pallas_call(
      _atom_body,
      grid=(N_BLOCKS,),
      in_specs=[
          pl.BlockSpec((BN_NODES, 9), lambda i: (i, 0)),
          pl.BlockSpec((1, 1, BN_NODES), lambda i: (i, 0, 0)),
          pl.BlockSpec((9, 64, D), lambda i: (0, 0, 0)),
          pl.BlockSpec((1, D), lambda i: (0, 0)),
      ],
      out_specs=[
          pl.BlockSpec((BN_NODES, D), lambda i: (i, 0)),
          pl.BlockSpec((G, D), lambda i: (0, 0)),
          pl.BlockSpec((G, 128), lambda i: (0, 0)),
      ],
      out_shape=[
          jax.ShapeDtypeStruct((N, D), jnp.float32),
          jax.ShapeDtypeStruct((G, D), jnp.float32),
          jax.ShapeDtypeStruct((G, 128), jnp.float32),
      ],
  )(x, batch3, atom_tables, vn_embed)


def _vn_body(gsum, vn, W1, b1, g1, bb1, W2, b2, g2, bb2, out):
  vt = gsum[...] + vn[...]
  u = jnp.dot(vt, W1[...], preferred_element_type=jnp.float32) + b1[...]
  u = jnp.maximum(g1[...] * (u * BN_INV) + bb1[...], 0.0)
  u = jnp.dot(u, W2[...], preferred_element_type=jnp.float32) + b2[...]
  u = jnp.maximum(g2[...] * (u * BN_INV) + bb2[...], 0.0)
  out[...] = u


def _vn_mlp(gsum, vn, W1, b1, g1, bb1, W2, b2, g2, bb2):
  return pl.pallas_call(
      _vn_body,
      out_shape=jax.ShapeDtypeStruct((G, D), jnp.float32),
  )(gsum, vn, W1, b1, g1, bb1, W2, b2, g2, bb2)


def _mlp_body(relu_out, add_vn, hin, agg, vn, batchb, eps, W1, b1, g1, bb1,
              W2, b2, g2, bb2, h_out, gsum):
  i = pl.program_id(0)
  pre = hin[...] * (1.0 + eps[0]) + jnp.concatenate(
      [agg[0], agg[1]], axis=1)
  t = jnp.dot(pre, W1[...], preferred_element_type=jnp.float32) + b1[...]
  t = jnp.maximum(g1[...] * (t * BN_INV) + bb1[...], 0.0)
  hc = jnp.dot(t, W2[...], preferred_element_type=jnp.float32) + b2[...]
  hc = g2[...] * (hc * BN_INV) + bb2[...]
  if relu_out:
    hc = jnp.maximum(hc, 0.0)
  oh = _onehot(batchb[0, 0, :], G)
  if add_vn:
    hc = hc + jnp.dot(oh, vn[...], preferred_element_type=jnp.float32)
  h_out[...] = hc

  @pl.when(i == 0)
  def _():
    gsum[...] = jnp.zeros_like(gsum)

  gsum[...] += lax.dot_general(oh, hc, (((0,), (0,)), ((), ())),
                               preferred_element_type=jnp.float32)


def _mlp(relu_out, add_vn, hin, agg, vn, batch3, eps, W1, b1, g1, bb1,
         W2, b2, g2, bb2):
  return pl.pallas_call(
      functools.partial(_mlp_body, relu_out, add_vn),
      grid=(N_BLOCKS,),
      in_specs=[
          pl.BlockSpec((BN_NODES, D), lambda i: (i, 0)),
          pl.BlockSpec((NC, BN_NODES, D // NC), lambda i: (0, i, 0)),
          pl.BlockSpec((G, D), lambda i: (0, 0)),
          pl.BlockSpec((1, 1, BN_NODES), lambda i: (i, 0, 0)),
          pl.BlockSpec(memory_space=pltpu.SMEM),
          pl.BlockSpec((D, H), lambda i: (0, 0)),
          pl.BlockSpec((1, H), lambda i: (0, 0)),
          pl.BlockSpec((1, H), lambda i: (0, 0)),
          pl.BlockSpec((1, H), lambda i: (0, 0)),
          pl.BlockSpec((H, D), lambda i: (0, 0)),
          pl.BlockSpec((1, D), lambda i: (0, 0)),
          pl.BlockSpec((1, D), lambda i: (0, 0)),
          pl.BlockSpec((1, D), lambda i: (0, 0)),
      ],
      out_specs=[
          pl.BlockSpec((BN_NODES, D), lambda i: (i, 0)),
          pl.BlockSpec((G, D), lambda i: (0, 0)),
      ],
      out_shape=[
          jax.ShapeDtypeStruct((N, D), jnp.float32),
          jax.ShapeDtypeStruct((G, D), jnp.float32),
      ],
  )(hin, agg, vn, batch3, eps, W1, b1, g1, bb1, W2, b2, g2, bb2)


def _pred_body(gsum, counts, W, b, out):
  cnt = jnp.maximum(counts[:, 0:1], 1.0)
  hg = gsum[...] / cnt
  out[...] = jnp.dot(hg, W[...], preferred_element_type=jnp.float32) + b[...]


def _pred(gsum, counts, W, b):
  return pl.pallas_call(
      _pred_body,
      out_shape=jax.ShapeDtypeStruct((G, T), jnp.float32),
  )(gsum, counts, W, b)


def kernel(x, edge_index, edge_attr, batch, atom_tables, bond_tables,
           conv_eps, conv_W1, conv_b1, conv_bn_g, conv_bn_b, conv_W2,
           conv_b2, node_bn_g, node_bn_b, vn_embed, vn_W1, vn_b1, vn_bn1_g,
           vn_bn1_b, vn_W2, vn_b2, vn_bn2_g, vn_bn2_b, pred_W, pred_b):
  src = edge_index[0]
  dst = edge_index[1]
  ea = edge_attr.T.reshape(3, E // 128, 128)
  batch3 = batch.reshape(N_BLOCKS, 1, BN_NODES)

  combo = _combo(ea[0], ea[1], ea[2]).reshape(E)
  ct = _ct(bond_tables)
  h_in, gsum, counts = _atom(x, batch3, atom_tables, vn_embed)
  vn = jnp.broadcast_to(vn_embed[0], (G, D))

  r2 = lambda a: a.reshape(1, -1)
  for l in range(L):
    agg = _sc_edge(h_in.reshape(NC * N, D // NC),
                   ct[l].reshape(NC * 512, D // NC), src, dst, combo)
    last = l == L - 1
    if not last:
      vn = _vn_mlp(gsum, vn, vn_W1[l], r2(vn_b1[l]), r2(vn_bn1_g[l]),
                   r2(vn_bn1_b[l]), vn_W2[l], r2(vn_b2[l]),
                   r2(vn_bn2_g[l]), r2(vn_bn2_b[l]))
    h_in, gsum = _mlp(
        not last, not last, h_in, agg, vn, batch3,
        (1.0 + 0.0) * conv_eps[l].reshape(1), conv_W1[l], r2(conv_b1[l]),
        r2(conv_bn_g[l]), r2(conv_bn_b[l]), conv_W2[l], r2(conv_b2[l]),
        r2(node_bn_g[l]), r2(node_bn_b[l]))

  return _pred(gsum, counts, pred_W, r2(pred_b))


# speedup vs baseline: 7.7843x; 2.0833x over previous
import functools

import jax
import jax.numpy as jnp
from jax import lax
from jax.experimental import pallas as pl
from jax.experimental.pallas import tpu as pltpu
from jax.experimental.pallas import tpu_sc as plsc

N = 10000
E = 160000
D = 256
H = 512
L = 5
G = 64
T = 128
BN_INV = 1.0 / (1.0 + 1e-5) ** 0.5

BN_NODES = 1000
N_BLOCKS = N // BN_NODES

NC = 2
NS = 16
EDGES_PER_TILE = E // NS
EB = 80
SB_EDGES = 2000
N_SB = EDGES_PER_TILE // SB_EDGES
NB_EDGE = SB_EDGES // EB
ROWS_PER_TILE = 624
ROWS_TAIL = N - NS * ROWS_PER_TILE


def _onehot(idx, num):
  iota = lax.broadcasted_iota(jnp.int32, (1, num), 1)
  return (idx[:, None] == iota).astype(jnp.float32)


def _sc_edge_body(h2, ct2, src, dst3d, combo, out, idx_h, idx_e, dst_t,
                  rows_h, rows_e, acc, sem_h0, sem_h1, sem_e0, sem_e1,
                  sem_s0, sem_s1):
  c = lax.axis_index("c")
  s = lax.axis_index("s")
  sem_h = (sem_h0, sem_h1)
  sem_e = (sem_e0, sem_e1)
  sem_s = (sem_s0, sem_s1)

  z = jnp.zeros((16,), jnp.float32)

  def zero_row(r, _):
    for j in range(8):
      rows_h[0, r, pl.ds(j * 16, 16)] = z
    return 0

  lax.fori_loop(0, EB, zero_row, 0)

  row0 = pl.multiple_of(s * ROWS_PER_TILE, 8)
  for j in range(ROWS_PER_TILE // EB):
    pltpu.sync_copy(rows_h.at[0], acc.at[pl.ds(row0 + j * EB, EB)])
  rem = ROWS_PER_TILE % EB
  pltpu.sync_copy(rows_h.at[0, pl.ds(0, rem)],
                  acc.at[pl.ds(row0 + (ROWS_PER_TILE // EB) * EB, rem)])

  @pl.when(s == NS - 1)
  def _():
    pltpu.sync_copy(rows_h.at[0, pl.ds(0, ROWS_TAIL)],
                    acc.at[pl.ds(NS * ROWS_PER_TILE, ROWS_TAIL)])

  plsc.subcore_barrier()

  def gather(i, slot):
    sl = pl.ds(pl.multiple_of(i * EB, 8), EB)
    pltpu.async_copy(h2.at[idx_h.at[sl]], rows_h.at[slot], sem_h[slot])
    pltpu.async_copy(ct2.at[idx_e.at[sl]], rows_e.at[slot], sem_e[slot])

  def wait_gather(slot):
    pltpu.make_async_copy(h2.at[idx_h.at[pl.ds(0, EB)]], rows_h.at[slot],
                          sem_h[slot]).wait()
    pltpu.make_async_copy(ct2.at[idx_e.at[pl.ds(0, EB)]], rows_e.at[slot],
                          sem_e[slot]).wait()

  def scatter(i, slot):
    pltpu.async_copy(rows_h.at[slot], acc.at[dst_t.at[i]], sem_s[slot],
                     add=True)

  def wait_scatter(slot):
    pltpu.make_async_copy(rows_h.at[slot], acc.at[dst_t.at[0]],
                          sem_s[slot]).wait()

  def compute(slot):
    def relu_row(r, _):
      for j in range(8):
        sl = pl.ds(j * 16, 16)
        rows_h[slot, r, sl] = jnp.maximum(
            rows_h[slot, r, sl] + rows_e[slot, r, sl], 0.0)
      return 0

    lax.fori_loop(0, EB, relu_row, 0)

  def superblock(sb, _):
    ebase = pl.multiple_of(s * EDGES_PER_TILE + sb * SB_EDGES, 8)
    pltpu.sync_copy(src.at[pl.ds(ebase, SB_EDGES)], idx_h)
    pltpu.sync_copy(combo.at[pl.ds(ebase, SB_EDGES)], idx_e)
    pltpu.sync_copy(dst3d.at[s * N_SB + sb], dst_t)

    def scale(k, _):
      sl = pl.ds(k * 16, 16)
      idx_h[sl] = idx_h[sl] * 2 + c
      idx_e[sl] = idx_e[sl] * 2 + c
      return 0

    lax.fori_loop(0, SB_EDGES // 16, scale, 0)

    gather(0, 0)

    def step(i2, _):
      for u in range(2):
        j = i2 * 2 + u
        slot = u

        @pl.when(j < NB_EDGE)
        def _():
          @pl.when(j >= 1)
          def _():
            wait_scatter(1 - slot)

          @pl.when(j + 1 < NB_EDGE)
          def _():
            gather(j + 1, 1 - slot)

          wait_gather(slot)
          compute(slot)
          scatter(j, slot)

      return 0

    lax.fori_loop(0, (NB_EDGE + 1) // 2, step, 0)
    wait_scatter(0)
    return 0

  lax.fori_loop(0, N_SB, superblock, 0)
  plsc.subcore_barrier()

  pltpu.sync_copy(acc.at[pl.ds(row0, ROWS_PER_TILE)],
                  out.at[c, pl.ds(row0, ROWS_PER_TILE)])

  @pl.when(s == NS - 1)
  def _():
    pltpu.sync_copy(acc.at[pl.ds(NS * ROWS_PER_TILE, ROWS_TAIL)],
                    out.at[c, pl.ds(NS * ROWS_PER_TILE, ROWS_TAIL)])


def _sc_edge(h2, ct2, src, dst2d, combo):
  fn = pl.kernel(
      _sc_edge_body,
      out_type=jax.ShapeDtypeStruct((NC, N, D // NC), jnp.float32),
      mesh=plsc.VectorSubcoreMesh(core_axis_name="c", subcore_axis_name="s"),
      scratch_types=[
          pltpu.VMEM((SB_EDGES,), jnp.int32),
          pltpu.VMEM((SB_EDGES,), jnp.int32),
          pltpu.VMEM((NB_EDGE, EB), jnp.int32),
          pltpu.VMEM((2, EB, D // NC), jnp.float32),
          pltpu.VMEM((2, EB, D // NC), jnp.float32),
          pltpu.VMEM_SHARED((N, D // NC), jnp.float32),
          pltpu.SemaphoreType.DMA,
          pltpu.SemaphoreType.DMA,
          pltpu.SemaphoreType.DMA,
          pltpu.SemaphoreType.DMA,
          pltpu.SemaphoreType.DMA,
          pltpu.SemaphoreType.DMA,
      ],
  )
  return fn(h2, ct2, src, dst2d, combo)


def _combo_body(ea0, ea1, ea2, out):
  out[...] = ea0[...] * 64 + ea1[...] * 8 + ea2[...]


def _combo(ea0, ea1, ea2):
  return pl.pallas_call(
      _combo_body,
      out_shape=jax.ShapeDtypeStruct((E // 128, 128), jnp.int32),
  )(ea0, ea1, ea2)


def _ct_body(bt, out):
  i512 = lax.broadcasted_iota(jnp.int32, (512,), 0)
  oh0 = _onehot(i512 // 64, 8)
  oh1 = _onehot((i512 // 8) % 8, 8)
  oh2 = _onehot(i512 % 8, 8)
  r = jnp.dot(oh0, bt[0, 0], preferred_element_type=jnp.float32)
  r += jnp.dot(oh1, bt[0, 1], preferred_element_type=jnp.float32)
  r += jnp.dot(oh2, bt[0, 2], preferred_element_type=jnp.float32)
  out[0] = r


def _ct(bond_tables):
  return pl.pallas_call(
      _ct_body,
      grid=(L,),
      in_specs=[pl.BlockSpec((1, 3, 8, D), lambda l: (l, 0, 0, 0))],
      out_specs=pl.BlockSpec((1, 512, D), lambda l: (l, 0, 0)),
      out_shape=jax.ShapeDtypeStruct((L, 512, D), jnp.float32),
  )(bond_tables)


def _atom_body(xb, batchb, tabs, vne, h_out, gsum, counts):
  i = pl.program_id(0)
  h = jnp.dot(_onehot(xb[:, 0], 64), tabs[0],
              preferred_element_type=jnp.float32)
  for j in range(1, 9):
    h += jnp.dot(_onehot(xb[:, j], 64), tabs[j],
                 preferred_element_type=jnp.float32)
  h += vne[...]
  h_out[...] = h
  oh = _onehot(batchb[0, 0, :], G)

  @pl.when(i == 0)
  def _():
    gsum[...] = jnp.zeros_like(gsum)
    counts[...] = jnp.zeros_like(counts)

  gsum[...] += lax.dot_general(oh, h, (((0,), (0,)), ((), ())),
                               preferred_element_type=jnp.float32)
  counts[...] += lax.dot_general(
      oh, jnp.ones((BN_NODES, 128), jnp.float32),
      (((0,), (0,)), ((), ())), preferred_element_type=jnp.float32)


def _atom(x, batch3, atom_tables, vn_embed):
  return pl.pallas_call(
      _atom_body,
      grid=(N_BLOCKS,),
      in_specs=[
          pl.BlockSpec((BN_NODES, 9), lambda i: (i, 0)),
          pl.BlockSpec((1, 1, BN_NODES), lambda i: (i, 0, 0)),
          pl.BlockSpec((9, 64, D), lambda i: (0, 0, 0)),
          pl.BlockSpec((1, D), lambda i: (0, 0)),
      ],
      out_specs=[
          pl.BlockSpec((BN_NODES, D), lambda i: (i, 0)),
          pl.BlockSpec((G, D), lambda i: (0, 0)),
          pl.BlockSpec((G, 128), lambda i: (0, 0)),
      ],
      out_shape=[
          jax.ShapeDtypeStruct((N, D), jnp.float32),
          jax.ShapeDtypeStruct((G, D), jnp.float32),
          jax.ShapeDtypeStruct((G, 128), jnp.float32),
      ],
  )(x, batch3, atom_tables, vn_embed)


def _vn_body(gsum, vn, W1, b1, g1, bb1, W2, b2, g2, bb2, out):
  vt = gsum[...] + vn[...]
  u = jnp.dot(vt, W1[...], preferred_element_type=jnp.float32) + b1[...]
  u = jnp.maximum(g1[...] * (u * BN_INV) + bb1[...], 0.0)
  u = jnp.dot(u, W2[...], preferred_element_type=jnp.float32) + b2[...]
  u = jnp.maximum(g2[...] * (u * BN_INV) + bb2[...], 0.0)
  out[...] = u


def _vn_mlp(gsum, vn, W1, b1, g1, bb1, W2, b2, g2, bb2):
  return pl.pallas_call(
      _vn_body,
      out_shape=jax.ShapeDtypeStruct((G, D), jnp.float32),
  )(gsum, vn, W1, b1, g1, bb1, W2, b2, g2, bb2)


def _mlp_body(relu_out, add_vn, hin, agg, vn, batchb, eps, W1, b1, g1, bb1,
              W2, b2, g2, bb2, h_out, gsum):
  i = pl.program_id(0)
  pre = hin[...] * (1.0 + eps[0]) + jnp.concatenate(
      [agg[0], agg[1]], axis=1)
  t = jnp.dot(pre, W1[...], preferred_element_type=jnp.float32) + b1[...]
  t = jnp.maximum(g1[...] * (t * BN_INV) + bb1[...], 0.0)
  hc = jnp.dot(t, W2[...], preferred_element_type=jnp.float32) + b2[...]
  hc = g2[...] * (hc * BN_INV) + bb2[...]
  if relu_out:
    hc = jnp.maximum(hc, 0.0)
  oh = _onehot(batchb[0, 0, :], G)
  if add_vn:
    hc = hc + jnp.dot(oh, vn[...], preferred_element_type=jnp.float32)
  h_out[...] = hc

  @pl.when(i == 0)
  def _():
    gsum[...] = jnp.zeros_like(gsum)

  gsum[...] += lax.dot_general(oh, hc, (((0,), (0,)), ((), ())),
                               preferred_element_type=jnp.float32)


def _mlp(relu_out, add_vn, hin, agg, vn, batch3, eps, W1, b1, g1, bb1,
         W2, b2, g2, bb2):
  return pl.pallas_call(
      functools.partial(_mlp_body, relu_out, add_vn),
      grid=(N_BLOCKS,),
      in_specs=[
          pl.BlockSpec((BN_NODES, D), lambda i: (i, 0)),
          pl.BlockSpec((NC, BN_NODES, D // NC), lambda i: (0, i, 0)),
          pl.BlockSpec((G, D), lambda i: (0, 0)),
          pl.BlockSpec((1, 1, BN_NODES), lambda i: (i, 0, 0)),
          pl.BlockSpec(memory_space=pltpu.SMEM),
          pl.BlockSpec((D, H), lambda i: (0, 0)),
          pl.BlockSpec((1, H), lambda i: (0, 0)),
          pl.BlockSpec((1, H), lambda i: (0, 0)),
          pl.BlockSpec((1, H), lambda i: (0, 0)),
          pl.BlockSpec((H, D), lambda i: (0, 0)),
          pl.BlockSpec((1, D), lambda i: (0, 0)),
          pl.BlockSpec((1, D), lambda i: (0, 0)),
          pl.BlockSpec((1, D), lambda i: (0, 0)),
      ],
      out_specs=[
          pl.BlockSpec((BN_NODES, D), lambda i: (i, 0)),
          pl.BlockSpec((G, D), lambda i: (0, 0)),
      ],
      out_shape=[
          jax.ShapeDtypeStruct((N, D), jnp.float32),
          jax.ShapeDtypeStruct((G, D), jnp.float32),
      ],
  )(hin, agg, vn, batch3, eps, W1, b1, g1, bb1, W2, b2, g2, bb2)


def _pred_body(gsum, counts, W, b, out):
  cnt = jnp.maximum(counts[:, 0:1], 1.0)
  hg = gsum[...] / cnt
  out[...] = jnp.dot(hg, W[...], preferred_element_type=jnp.float32) + b[...]


def _pred(gsum, counts, W, b):
  return pl.pallas_call(
      _pred_body,
      out_shape=jax.ShapeDtypeStruct((G, T), jnp.float32),
  )(gsum, counts, W, b)


def kernel(x, edge_index, edge_attr, batch, atom_tables, bond_tables,
           conv_eps, conv_W1, conv_b1, conv_bn_g, conv_bn_b, conv_W2,
           conv_b2, node_bn_g, node_bn_b, vn_embed, vn_W1, vn_b1, vn_bn1_g,
           vn_bn1_b, vn_W2, vn_b2, vn_bn2_g, vn_bn2_b, pred_W, pred_b):
  src = edge_index[0]
  dst2d = edge_index[1].reshape(NS * N_SB, NB_EDGE, EB)
  ea = edge_attr.T.reshape(3, E // 128, 128)
  batch3 = batch.reshape(N_BLOCKS, 1, BN_NODES)

  combo = _combo(ea[0], ea[1], ea[2]).reshape(E)
  ct = _ct(bond_tables)
  h_in, gsum, counts = _atom(x, batch3, atom_tables, vn_embed)
  vn = jnp.broadcast_to(vn_embed[0], (G, D))

  r2 = lambda a: a.reshape(1, -1)
  for l in range(L):
    agg = _sc_edge(h_in.reshape(NC * N, D // NC),
                   ct[l].reshape(NC * 512, D // NC), src, dst2d, combo)
    last = l == L - 1
    if not last:
      vn = _vn_mlp(gsum, vn, vn_W1[l], r2(vn_b1[l]), r2(vn_bn1_g[l]),
                   r2(vn_bn1_b[l]), vn_W2[l], r2(vn_b2[l]),
                   r2(vn_bn2_g[l]), r2(vn_bn2_b[l]))
    h_in, gsum = _mlp(
        not last, not last, h_in, agg, vn, batch3,
        (1.0 + 0.0) * conv_eps[l].reshape(1), conv_W1[l], r2(conv_b1[l]),
        r2(conv_bn_g[l]), r2(conv_bn_b[l]), conv_W2[l], r2(conv_b2[l]),
        r2(node_bn_g[l]), r2(node_bn_b[l]))

  return _pred(gsum, counts, pred_W, r2(pred_b))
